# Initial kernel scaffold; baseline (speedup 1.0000x reference)
import jax
import jax.numpy as jnp
from jax.experimental import pallas as pl


def kernel(action_indices, table):
    raise NotImplementedError("write your pallas kernel here")


# speedup vs baseline: 2.4213x; 2.4213x over previous
import functools

import jax
import jax.numpy as jnp
from jax import lax
from jax.experimental import pallas as pl
from jax.experimental.pallas import tpu as pltpu
from jax.experimental.pallas import tpu_sc as plsc

_info = plsc.get_sparse_core_info()
_NC, _NS = _info.num_cores, _info.num_subcores
_NW = _NC * _NS

_C = 128


@functools.cache
def _make_lookup(B, D):
    bpw = B // _NW
    nchunks = bpw // _C
    npairs = nchunks // 2
    mesh = plsc.VectorSubcoreMesh(core_axis_name="c", subcore_axis_name="s")

    @functools.partial(
        pl.kernel,
        out_type=jax.ShapeDtypeStruct((B, D), jnp.float32),
        mesh=mesh,
        scratch_types=[
            pltpu.VMEM((bpw,), jnp.int32),
            pltpu.VMEM((_C, D), jnp.float32),
            pltpu.VMEM((_C, D), jnp.float32),
            pltpu.SemaphoreType.DMA,
            pltpu.SemaphoreType.DMA,
            pltpu.SemaphoreType.DMA,
            pltpu.SemaphoreType.DMA,
        ],
    )
    def lookup(idx_hbm, table_hbm, out_hbm, idx_v, buf0, buf1, g0, g1, o0, o1):
        wid = lax.axis_index("s") * _NC + lax.axis_index("c")
        base = wid * bpw
        pltpu.sync_copy(idx_hbm.at[pl.ds(base, bpw)], idx_v)
        bufs = (buf0, buf1)
        gsems = (g0, g1)
        osems = (o0, o1)

        def start_gather(ch, b):
            pltpu.async_copy(
                table_hbm.at[idx_v.at[pl.ds(ch * _C, _C)]], bufs[b], gsems[b]
            )

        start_gather(0, 0)
        start_gather(1, 1)

        def pair(p, carry):
            for b in range(2):
                ch = 2 * p + b
                pltpu.make_async_copy(
                    table_hbm.at[idx_v.at[pl.ds(ch * _C, _C)]], bufs[b], gsems[b]
                ).wait()
                out_done = pltpu.async_copy(
                    bufs[b], out_hbm.at[pl.ds(base + ch * _C, _C)], osems[b]
                )
                out_done.wait()

                @pl.when(ch + 2 < nchunks)
                def _():
                    start_gather(ch + 2, b)

            return carry

        lax.fori_loop(0, npairs, pair, None)

    return lookup


def kernel(action_indices, table):
    n, s = action_indices.shape
    B = n * s
    D = table.shape[1]
    flat_idx = action_indices.reshape((B,)).astype(jnp.int32)
    out = _make_lookup(B, D)(flat_idx, table)
    return out.reshape((n, s, D))


# speedup vs baseline: 3.6756x; 1.5180x over previous
import functools

import jax
import jax.numpy as jnp
from jax import lax
from jax.experimental import pallas as pl
from jax.experimental.pallas import tpu as pltpu
from jax.experimental.pallas import tpu_sc as plsc

_info = plsc.get_sparse_core_info()
_NC, _NS = _info.num_cores, _info.num_subcores
_NW = _NC * _NS

_S_PAD = 56
_TILE = 8


@functools.cache
def _make_lookup(N, S, D):
    ipw = N // _NW
    s_main = (S // _TILE) * _TILE
    s_tail = S - s_main
    mesh = plsc.VectorSubcoreMesh(core_axis_name="c", subcore_axis_name="s")

    ntail = ipw * s_tail
    assert ntail % 32 == 0

    @functools.partial(
        pl.kernel,
        out_type=(
            jax.ShapeDtypeStruct((N, S, D), jnp.float32),
            jax.ShapeDtypeStruct((N * s_tail, D), jnp.float32),
        ),
        mesh=mesh,
        compiler_params=pltpu.CompilerParams(needs_layout_passes=False),
        scratch_types=[
            pltpu.VMEM((ipw * _S_PAD,), jnp.int32),
            pltpu.VMEM((ntail,), jnp.int32),
            pltpu.VMEM((s_main, D), jnp.float32),
            pltpu.VMEM((s_main, D), jnp.float32),
            pltpu.VMEM((ntail // 2, D), jnp.float32),
            pltpu.VMEM((ntail // 2, D), jnp.float32),
            pltpu.SemaphoreType.DMA,
            pltpu.SemaphoreType.DMA,
            pltpu.SemaphoreType.DMA,
            pltpu.SemaphoreType.DMA,
            pltpu.SemaphoreType.DMA,
        ],
    )
    def lookup(
        idxp_hbm, table_hbm, out_hbm, tail_hbm,
        idx_v, tidx_v, buf0, buf1, tbuf0, tbuf1, g0, g1, o0, o1, tsem,
    ):
        wid = lax.axis_index("s") * _NC + lax.axis_index("c")
        item0 = wid * ipw
        pltpu.sync_copy(idxp_hbm.at[pl.ds(item0 * _S_PAD, ipw * _S_PAD)], idx_v)
        bufs = (buf0, buf1)
        gsems = (g0, g1)
        osems = (o0, o1)

        def start_gather(it, b):
            pltpu.async_copy(
                table_hbm.at[idx_v.at[pl.ds(it * _S_PAD, s_main)]], bufs[b], gsems[b]
            )

        start_gather(0, 0)
        start_gather(1, 1)

        def pair(p, carry):
            for b in range(2):
                it = 2 * p + b
                pltpu.make_async_copy(
                    table_hbm.at[idx_v.at[pl.ds(it * _S_PAD, s_main)]], bufs[b], gsems[b]
                ).wait()
                out_done = pltpu.async_copy(
                    bufs[b],
                    out_hbm.at[item0 + it].at[pl.ds(0, s_main)],
                    osems[b],
                )
                out_done.wait()

                @pl.when(it + 2 < ipw)
                def _():
                    start_gather(it + 2, b)

            return carry

        lane = lax.iota(jnp.int32, 16)
        for k in range(ntail // 16):
            q = lane + (16 * k)
            p = lax.shift_right_logical(q, 1) * _S_PAD + s_main + (q & 1)
            tidx_v[pl.ds(16 * k, 16)] = plsc.load_gather(idx_v, [p])

        pltpu.async_copy(table_hbm.at[tidx_v.at[pl.ds(0, ntail // 2)]], tbuf0, tsem)
        pltpu.async_copy(
            table_hbm.at[tidx_v.at[pl.ds(ntail // 2, ntail // 2)]], tbuf1, tsem
        )

        lax.fori_loop(0, ipw // 2, pair, None)

        base_t = item0 * s_tail
        pltpu.make_async_copy(
            table_hbm.at[tidx_v.at[pl.ds(0, ntail // 2)]], tbuf0, tsem
        ).wait()
        pltpu.make_async_copy(
            table_hbm.at[tidx_v.at[pl.ds(ntail // 2, ntail // 2)]], tbuf1, tsem
        ).wait()
        pltpu.sync_copy(tbuf0, tail_hbm.at[pl.ds(base_t, ntail // 2)])
        pltpu.sync_copy(tbuf1, tail_hbm.at[pl.ds(base_t + ntail // 2, ntail // 2)])

    return lookup


def kernel(action_indices, table):
    n, s = action_indices.shape
    D = table.shape[1]
    s_main = (s // _TILE) * _TILE
    idx_pad = jnp.pad(
        action_indices.astype(jnp.int32), ((0, 0), (0, _S_PAD - s))
    ).reshape((n * _S_PAD,))
    main, tails = _make_lookup(n, s, D)(idx_pad, table)
    return lax.dynamic_update_slice(
        main, tails.reshape((n, s - s_main, D)), (0, s_main, 0)
    )


# speedup vs baseline: 7.4612x; 2.0299x over previous
import functools

import jax
import jax.numpy as jnp
from jax import lax
from jax.experimental import pallas as pl
from jax.experimental.pallas import tpu as pltpu
from jax.experimental.pallas import tpu_sc as plsc

_info = plsc.get_sparse_core_info()
_NC, _NS = _info.num_cores, _info.num_subcores
_NW = _NC * _NS


@functools.cache
def _make_lookup(N, S, D):
    ipw = N // _NW
    mesh = plsc.VectorSubcoreMesh(core_axis_name="c", subcore_axis_name="s")
    assert S % 2 == 0 and ipw % 8 == 0 and ipw <= 128

    @functools.partial(
        pl.kernel,
        out_type=jax.ShapeDtypeStruct((S, N, D), jnp.float32),
        mesh=mesh,
        scratch_types=[
            pltpu.VMEM((S, ipw), jnp.int32),
            pltpu.VMEM((ipw, D), jnp.float32),
            pltpu.VMEM((ipw, D), jnp.float32),
            pltpu.SemaphoreType.DMA,
            pltpu.SemaphoreType.DMA,
            pltpu.SemaphoreType.DMA,
            pltpu.SemaphoreType.DMA,
        ],
    )
    def lookup(idxt_hbm, table_hbm, out_hbm, idx_v, buf0, buf1, g0, g1, o0, o1):
        wid = lax.axis_index("s") * _NC + lax.axis_index("c")
        col0 = wid * ipw
        pltpu.sync_copy(idxt_hbm.at[:, pl.ds(col0, ipw)], idx_v)
        bufs = (buf0, buf1)
        gsems = (g0, g1)
        osems = (o0, o1)

        def start_gather(j, b):
            pltpu.async_copy(table_hbm.at[idx_v.at[j]], bufs[b], gsems[b])

        start_gather(0, 0)
        start_gather(1, 1)

        def pair(p, carry):
            for b in range(2):
                j = 2 * p + b
                pltpu.make_async_copy(
                    table_hbm.at[idx_v.at[j]], bufs[b], gsems[b]
                ).wait()
                out_done = pltpu.async_copy(
                    bufs[b], out_hbm.at[j].at[pl.ds(col0, ipw)], osems[b]
                )
                out_done.wait()

                @pl.when(j + 2 < S)
                def _():
                    start_gather(j + 2, b)

            return carry

        lax.fori_loop(0, S // 2, pair, None)

    return lookup


def kernel(action_indices, table):
    n, s = action_indices.shape
    D = table.shape[1]
    idx_t = jnp.transpose(action_indices.astype(jnp.int32))
    out_t = _make_lookup(n, s, D)(idx_t, table)
    return jnp.transpose(out_t, (1, 0, 2))


# speedup vs baseline: 7.4838x; 1.0030x over previous
import functools

import jax
import jax.numpy as jnp
from jax import lax
from jax.experimental import pallas as pl
from jax.experimental.pallas import tpu as pltpu
from jax.experimental.pallas import tpu_sc as plsc

_info = plsc.get_sparse_core_info()
_NC, _NS = _info.num_cores, _info.num_subcores
_NW = _NC * _NS


@functools.cache
def _make_lookup(N, S, D):
    ipw = N // _NW
    mesh = plsc.VectorSubcoreMesh(core_axis_name="c", subcore_axis_name="s")
    assert S % 2 == 0 and ipw % 8 == 0 and ipw <= 128

    half = ipw // 2
    nchunks = 2 * S

    @functools.partial(
        pl.kernel,
        out_type=jax.ShapeDtypeStruct((S, N, D), jnp.float32),
        mesh=mesh,
        scratch_types=[
            pltpu.VMEM((S, ipw), jnp.int32),
            pltpu.VMEM((half, D), jnp.float32),
            pltpu.VMEM((half, D), jnp.float32),
            pltpu.VMEM((half, D), jnp.float32),
            pltpu.VMEM((half, D), jnp.float32),
            pltpu.SemaphoreType.DMA,
            pltpu.SemaphoreType.DMA,
            pltpu.SemaphoreType.DMA,
            pltpu.SemaphoreType.DMA,
            pltpu.SemaphoreType.DMA,
            pltpu.SemaphoreType.DMA,
            pltpu.SemaphoreType.DMA,
            pltpu.SemaphoreType.DMA,
        ],
    )
    def lookup(
        idxt_hbm, table_hbm, out_hbm, idx_v,
        buf0, buf1, buf2, buf3, g0, g1, g2, g3, o0, o1, o2, o3,
    ):
        wid = lax.axis_index("s") * _NC + lax.axis_index("c")
        col0 = wid * ipw
        pltpu.sync_copy(idxt_hbm.at[:, pl.ds(col0, ipw)], idx_v)
        bufs = (buf0, buf1, buf2, buf3)
        gsems = (g0, g1, g2, g3)
        osems = (o0, o1, o2, o3)

        def idx_slice(j, h):
            return idx_v.at[j].at[pl.ds(h * half, half)]

        def out_slice(j, h):
            return out_hbm.at[j].at[pl.ds(col0 + h * half, half)]

        def start_gather(j, h, b):
            pltpu.async_copy(table_hbm.at[idx_slice(j, h)], bufs[b], gsems[b])

        def wait_gather(j, h, b):
            pltpu.make_async_copy(
                table_hbm.at[idx_slice(j, h)], bufs[b], gsems[b]
            ).wait()

        def start_out(j, h, b):
            pltpu.async_copy(bufs[b], out_slice(j, h), osems[b])

        def wait_out(j, h, b):
            pltpu.make_async_copy(bufs[b], out_slice(j, h), osems[b]).wait()

        start_gather(0, 0, 0)
        start_gather(0, 1, 1)
        start_gather(1, 0, 2)

        def quad(p, carry):
            for b in range(4):
                j = 2 * p + (b // 2)
                h = b % 2
                wait_gather(j, h, b)
                start_out(j, h, b)
                b2 = (b + 3) % 4
                jn, hn = 2 * p + ((b + 3) // 2), (b + 3) % 2
                jw, hw = 2 * p + ((b - 1) // 2), (b - 1) % 2

                if b == 0:

                    @pl.when(p == 0)
                    def _():
                        start_gather(jn, hn, b2)

                    @pl.when(jnp.logical_and(p > 0, 2 * jn + hn < nchunks))
                    def _():
                        wait_out(jw, hw, b2)
                        start_gather(jn, hn, b2)
                else:

                    @pl.when(2 * jn + hn < nchunks)
                    def _():
                        wait_out(jw, hw, b2)
                        start_gather(jn, hn, b2)

            return carry

        lax.fori_loop(0, nchunks // 4, quad, None)
        for c in range(nchunks - 4, nchunks):
            wait_out(c // 2, c % 2, c % 4)

    return lookup


def kernel(action_indices, table):
    n, s = action_indices.shape
    D = table.shape[1]
    idx_t = jnp.transpose(action_indices.astype(jnp.int32))
    out_t = _make_lookup(n, s, D)(idx_t, table)
    return jnp.transpose(out_t, (1, 0, 2))
